# Initial kernel scaffold; baseline (speedup 1.0000x reference)
import jax
import jax.numpy as jnp
from jax.experimental import pallas as pl


def kernel(entity_emb, user_emb, intent_emb, edge_index, edge_type, interact_mat, r_emb):
    raise NotImplementedError("write your pallas kernel here")


# speedup vs baseline: 3.7526x; 3.7526x over previous
import functools

import jax
import jax.numpy as jnp
from jax import lax
from jax.experimental import pallas as pl
from jax.experimental.pallas import tpu as pltpu
from jax.experimental.pallas import tpu_sc as plsc

_N_ENT = 10000
_N_ENT_PAD = 10240
_EMB = 128
_HALF = 64
_N_EDGE = 320000
_NT = 16
_EPT = _N_EDGE // _NT
_CHUNK = 80
_NCHUNK = _EPT // _CHUNK
_CBLK = 25
_NBLK = _NCHUNK // _CBLK
_ROWS_PER_TILE = _N_ENT_PAD // _NT
_WCHUNK = 128
_NWCHUNK = _ROWS_PER_TILE // _WCHUNK


def _make_edge_kernel():
    mesh = plsc.VectorSubcoreMesh(core_axis_name="c", subcore_axis_name="s")

    @functools.partial(
        pl.kernel,
        out_type=(
            jax.ShapeDtypeStruct((2, _N_ENT_PAD, _HALF), jnp.float32),
            jax.ShapeDtypeStruct((2, _N_ENT_PAD), jnp.float32),
        ),
        mesh=mesh,
        scratch_types=[
            pltpu.VMEM((_CBLK, _CHUNK), jnp.int32),
            pltpu.VMEM((_CBLK, _CHUNK), jnp.int32),
            pltpu.VMEM((_CBLK, _CHUNK), jnp.int32),
            pltpu.VMEM((_CHUNK, _HALF), jnp.float32),
            pltpu.VMEM((16, _HALF), jnp.float32),
            pltpu.VMEM((_CHUNK,), jnp.float32),
            pltpu.VMEM((_WCHUNK, _HALF), jnp.float32),
            pltpu.VMEM((_ROWS_PER_TILE,), jnp.float32),
            pltpu.VMEM_SHARED((_N_ENT_PAD, _HALF), jnp.float32),
            pltpu.VMEM_SHARED((_N_ENT_PAD,), jnp.float32),
            pltpu.SemaphoreType.DMA,
        ],
        compiler_params=pltpu.CompilerParams(use_tc_tiling_on_sc=False),
    )
    def edge_kernel(head_hbm, tail_hbm, rel_hbm, ent_hbm, remb_hbm,
                    psum_hbm, pcnt_hbm,
                    hd_v, tl_v, rl_v, rows_v, remb_v, ones_v, stage_v, cnt_v,
                    acc_sum, acc_cnt, sem):
        c = lax.axis_index("c")
        s = lax.axis_index("s")
        base = s * _ROWS_PER_TILE

        pltpu.sync_copy(remb_hbm.at[c], remb_v)

        for i in range(_CHUNK // 16):
            ones_v[pl.ds(i * 16, 16)] = jnp.ones((16,), jnp.float32)

        def _zero_stage(i, _):
            for j in range(_HALF // 16):
                stage_v[i, pl.ds(j * 16, 16)] = jnp.zeros((16,), jnp.float32)
            return 0
        lax.fori_loop(0, _WCHUNK, _zero_stage, 0)

        def _zero_cnt(i, _):
            cnt_v[pl.ds(i * 16, 16)] = jnp.zeros((16,), jnp.float32)
            return 0
        lax.fori_loop(0, _ROWS_PER_TILE // 16, _zero_cnt, 0)

        for i in range(_NWCHUNK):
            pltpu.sync_copy(stage_v, acc_sum.at[pl.ds(base + i * _WCHUNK, _WCHUNK)])
        pltpu.sync_copy(cnt_v, acc_cnt.at[pl.ds(base, _ROWS_PER_TILE)])

        plsc.subcore_barrier()

        def _blk(b, _):
            pltpu.sync_copy(head_hbm.at[s, b], hd_v)
            pltpu.sync_copy(tail_hbm.at[s, b], tl_v)
            pltpu.sync_copy(rel_hbm.at[s, b], rl_v)

            def _chunk(k, _):
                pltpu.async_copy(ent_hbm.at[c].at[tl_v.at[k]], rows_v, sem).wait()

                def _group(g, _):
                    relv = rl_v[k, pl.ds(g * 16, 16)]
                    e0 = g * 16
                    for l in range(16):
                        r = relv[l]
                        e = e0 + l
                        for j in range(_HALF // 16):
                            sl = pl.ds(j * 16, 16)
                            rows_v[e, sl] = rows_v[e, sl] * remb_v[r, sl]
                    return 0
                lax.fori_loop(0, _CHUNK // 16, _group, 0)

                pltpu.sync_copy(rows_v, acc_sum.at[hd_v.at[k]], add=True)
                pltpu.sync_copy(ones_v, acc_cnt.at[hd_v.at[k]], add=True)
                return 0
            lax.fori_loop(0, _CBLK, _chunk, 0)
            return 0
        lax.fori_loop(0, _NBLK, _blk, 0)

        plsc.subcore_barrier()

        for i in range(_NWCHUNK):
            off = base + i * _WCHUNK
            pltpu.sync_copy(acc_sum.at[pl.ds(off, _WCHUNK)], stage_v)
            pltpu.sync_copy(stage_v, psum_hbm.at[c, pl.ds(off, _WCHUNK)])
        pltpu.sync_copy(acc_cnt.at[pl.ds(base, _ROWS_PER_TILE)], cnt_v)
        pltpu.sync_copy(cnt_v, pcnt_hbm.at[c, pl.ds(base, _ROWS_PER_TILE)])

    return edge_kernel


_edge_agg = _make_edge_kernel()


def _finalize_body(ps_ref, pc_ref, out_ref):
    sums = jnp.concatenate([ps_ref[0], ps_ref[1]], axis=1)
    out_ref[...] = sums / jnp.maximum(pc_ref[...], 1.0)


def _finalize(psum, pcnt):
    blk = 2048
    grid = _N_ENT_PAD // blk
    return pl.pallas_call(
        _finalize_body,
        grid=(grid,),
        in_specs=[
            pl.BlockSpec((2, blk, _HALF), lambda i: (0, i, 0)),
            pl.BlockSpec((blk, 1), lambda i: (i, 0)),
        ],
        out_specs=pl.BlockSpec((blk, _EMB), lambda i: (i, 0)),
        out_shape=jax.ShapeDtypeStruct((_N_ENT_PAD, _EMB), jnp.float32),
    )(psum, pcnt[0].reshape(_N_ENT_PAD, 1))


def _intent_vec(row, sub):
    logits = jnp.sum(row * sub, axis=1, keepdims=True)
    m = jnp.max(logits, axis=0, keepdims=True)
    e = jnp.exp(logits - m)
    att = e / jnp.sum(e, axis=0, keepdims=True)
    return jnp.sum(att * sub, axis=0, keepdims=True) / sub.shape[0]


def _user_body(u_ref, im_ref, ent_ref, it_ref, r_ref, out_ref):
    it = it_ref[...]
    r = r_ref[...]
    parts = [
        _intent_vec(it[0:1], r),
        _intent_vec(it[1:2], r[0:4]),
        _intent_vec(it[2:3], r[4:8]),
        _intent_vec(it[3:4], r[8:12]),
        _intent_vec(it[4:5], r[12:16]),
    ]
    all_intent = jnp.concatenate(parts, axis=0)
    new_intent = (all_intent + it) * 0.5

    u = u_ref[...]
    score_ = jax.lax.dot_general(
        u, new_intent, (((1,), (1,)), ((), ())),
        preferred_element_type=jnp.float32)
    sm = jnp.max(score_, axis=1, keepdims=True)
    se = jnp.exp(score_ - sm)
    score = se / jnp.sum(se, axis=1, keepdims=True)

    wvec = jax.lax.dot_general(
        score, new_intent, (((1,), (0,)), ((), ())),
        preferred_element_type=jnp.float32)

    agg = jax.lax.dot_general(
        im_ref[...], ent_ref[...], (((1,), (0,)), ((), ())),
        preferred_element_type=jnp.float32)
    out_ref[...] = agg * (1.0 + wvec)


def _user_agg(user_emb, interact_mat, entity_emb, intent_emb, r_emb):
    n_users = user_emb.shape[0]
    blk = 512
    grid = n_users // blk
    return pl.pallas_call(
        _user_body,
        grid=(grid,),
        in_specs=[
            pl.BlockSpec((blk, _EMB), lambda i: (i, 0)),
            pl.BlockSpec((blk, _N_ENT), lambda i: (i, 0)),
            pl.BlockSpec((_N_ENT, _EMB), lambda i: (0, 0)),
            pl.BlockSpec((5, _EMB), lambda i: (0, 0)),
            pl.BlockSpec((16, _EMB), lambda i: (0, 0)),
        ],
        out_specs=pl.BlockSpec((blk, _EMB), lambda i: (i, 0)),
        out_shape=jax.ShapeDtypeStruct((n_users, _EMB), jnp.float32),
    )(user_emb, interact_mat, entity_emb, intent_emb, r_emb)


def kernel(entity_emb, user_emb, intent_emb, edge_index, edge_type, interact_mat, r_emb):
    head = edge_index[0].astype(jnp.int32).reshape(_NT, _NBLK, _CBLK, _CHUNK)
    tail = edge_index[1].astype(jnp.int32).reshape(_NT, _NBLK, _CBLK, _CHUNK)
    rel = ((edge_type.astype(jnp.int32) - 1) & 15).reshape(_NT, _NBLK, _CBLK, _CHUNK)

    ent_halves = jnp.stack([entity_emb[:, :_HALF], entity_emb[:, _HALF:]])
    remb_halves = jnp.stack([r_emb[:, :_HALF], r_emb[:, _HALF:]])

    psum, pcnt = _edge_agg(head, tail, rel, ent_halves, remb_halves)
    entity_agg = _finalize(psum, pcnt)[:_N_ENT]
    user_agg = _user_agg(user_emb, interact_mat, entity_emb, intent_emb, r_emb)
    return entity_agg, user_agg


# speedup vs baseline: 4.2676x; 1.1372x over previous
import functools

import jax
import jax.numpy as jnp
from jax import lax
from jax.experimental import pallas as pl
from jax.experimental.pallas import tpu as pltpu
from jax.experimental.pallas import tpu_sc as plsc

_N_ENT = 10000
_N_ENT_PAD = 10240
_EMB = 128
_HALF = 64
_N_EDGE = 320000
_NT = 16
_EPT = _N_EDGE // _NT
_CHUNK = 128
_NPROC = 158
_NTOT = 160
_EPT_PAD = _NTOT * _CHUNK
_ROWS_PER_TILE = _N_ENT_PAD // _NT
_WCHUNK = 128
_NWCHUNK = _ROWS_PER_TILE // _WCHUNK


def _make_edge_kernel():
    mesh = plsc.VectorSubcoreMesh(core_axis_name="c", subcore_axis_name="s")

    @functools.partial(
        pl.kernel,
        out_type=(
            jax.ShapeDtypeStruct((2, _N_ENT_PAD, _HALF), jnp.float32),
            jax.ShapeDtypeStruct((2, _N_ENT_PAD), jnp.float32),
        ),
        mesh=mesh,
        scratch_types=[
            pltpu.VMEM((_NTOT, _CHUNK), jnp.int32),
            pltpu.VMEM((_NTOT, _CHUNK), jnp.int32),
            pltpu.VMEM((_NTOT, _CHUNK), jnp.int32),
            pltpu.VMEM((_CHUNK, _HALF), jnp.float32),
            pltpu.VMEM((_CHUNK, _HALF), jnp.float32),
            pltpu.VMEM((16, _HALF), jnp.float32),
            pltpu.VMEM((_CHUNK,), jnp.float32),
            pltpu.VMEM((_WCHUNK, _HALF), jnp.float32),
            pltpu.VMEM((_ROWS_PER_TILE,), jnp.float32),
            pltpu.VMEM_SHARED((_N_ENT_PAD, _HALF), jnp.float32),
            pltpu.VMEM_SHARED((_N_ENT_PAD,), jnp.float32),
            pltpu.SemaphoreType.DMA,
            pltpu.SemaphoreType.DMA,
        ],
        compiler_params=pltpu.CompilerParams(use_tc_tiling_on_sc=False),
    )
    def edge_kernel(head_hbm, tail_hbm, rel_hbm, ent_hbm, remb_hbm,
                    psum_hbm, pcnt_hbm,
                    hd_v, tl_v, rl_v, rows_a, rows_b, remb_v, ones_v, stage_v,
                    cnt_v, acc_sum, acc_cnt, sem_a, sem_b):
        c = lax.axis_index("c")
        s = lax.axis_index("s")
        base = s * _ROWS_PER_TILE

        pltpu.sync_copy(remb_hbm.at[c], remb_v)
        pltpu.sync_copy(head_hbm.at[s], hd_v)
        pltpu.sync_copy(tail_hbm.at[s], tl_v)
        pltpu.sync_copy(rel_hbm.at[s], rl_v)

        for i in range(_CHUNK // 16):
            ones_v[pl.ds(i * 16, 16)] = jnp.ones((16,), jnp.float32)

        def _zero_stage(i, _):
            for j in range(_HALF // 16):
                stage_v[i, pl.ds(j * 16, 16)] = jnp.zeros((16,), jnp.float32)
            return 0
        lax.fori_loop(0, _WCHUNK, _zero_stage, 0)

        def _zero_cnt(i, _):
            cnt_v[pl.ds(i * 16, 16)] = jnp.zeros((16,), jnp.float32)
            return 0
        lax.fori_loop(0, _ROWS_PER_TILE // 16, _zero_cnt, 0)

        for i in range(_NWCHUNK):
            pltpu.sync_copy(stage_v, acc_sum.at[pl.ds(base + i * _WCHUNK, _WCHUNK)])
        pltpu.sync_copy(cnt_v, acc_cnt.at[pl.ds(base, _ROWS_PER_TILE)])

        plsc.subcore_barrier()

        pltpu.async_copy(ent_hbm.at[c].at[tl_v.at[0]], rows_a, sem_a)
        pltpu.async_copy(ent_hbm.at[c].at[tl_v.at[1]], rows_b, sem_b)

        def _pair(k2, _):
            for buf, sem, off in ((rows_a, sem_a, 0), (rows_b, sem_b, 1)):
                kc = 2 * k2 + off
                pltpu.make_async_copy(ent_hbm.at[c].at[tl_v.at[kc]], buf, sem).wait()

                def _group(g, _, buf=buf, kc=kc):
                    relv = rl_v[kc, pl.ds(g * 16, 16)]
                    e0 = g * 16
                    for l in range(16):
                        r = relv[l]
                        e = e0 + l
                        for j in range(_HALF // 16):
                            sl = pl.ds(j * 16, 16)
                            buf[e, sl] = buf[e, sl] * remb_v[r, sl]
                    return 0
                lax.fori_loop(0, _CHUNK // 16, _group, 0)

                pltpu.sync_copy(buf, acc_sum.at[hd_v.at[kc]], add=True)
                pltpu.sync_copy(ones_v, acc_cnt.at[hd_v.at[kc]], add=True)
                pltpu.async_copy(ent_hbm.at[c].at[tl_v.at[kc + 2]], buf, sem)
            return 0
        lax.fori_loop(0, _NPROC // 2, _pair, 0)

        pltpu.make_async_copy(ent_hbm.at[c].at[tl_v.at[_NPROC]], rows_a, sem_a).wait()
        pltpu.make_async_copy(ent_hbm.at[c].at[tl_v.at[_NPROC + 1]], rows_b, sem_b).wait()

        plsc.subcore_barrier()

        for i in range(_NWCHUNK):
            off = base + i * _WCHUNK
            pltpu.sync_copy(acc_sum.at[pl.ds(off, _WCHUNK)], stage_v)
            pltpu.sync_copy(stage_v, psum_hbm.at[c, pl.ds(off, _WCHUNK)])
        pltpu.sync_copy(acc_cnt.at[pl.ds(base, _ROWS_PER_TILE)], cnt_v)
        pltpu.sync_copy(cnt_v, pcnt_hbm.at[c, pl.ds(base, _ROWS_PER_TILE)])

    return edge_kernel


_edge_agg = _make_edge_kernel()


def _finalize_body(ps_ref, pc_ref, out_ref):
    sums = jnp.concatenate([ps_ref[0], ps_ref[1]], axis=1)
    out_ref[...] = sums / jnp.maximum(pc_ref[...], 1.0)


def _finalize(psum, pcnt):
    blk = 2048
    grid = _N_ENT_PAD // blk
    return pl.pallas_call(
        _finalize_body,
        grid=(grid,),
        in_specs=[
            pl.BlockSpec((2, blk, _HALF), lambda i: (0, i, 0)),
            pl.BlockSpec((blk, 1), lambda i: (i, 0)),
        ],
        out_specs=pl.BlockSpec((blk, _EMB), lambda i: (i, 0)),
        out_shape=jax.ShapeDtypeStruct((_N_ENT_PAD, _EMB), jnp.float32),
    )(psum, pcnt[0].reshape(_N_ENT_PAD, 1))


def _intent_vec(row, sub):
    logits = jnp.sum(row * sub, axis=1, keepdims=True)
    m = jnp.max(logits, axis=0, keepdims=True)
    e = jnp.exp(logits - m)
    att = e / jnp.sum(e, axis=0, keepdims=True)
    return jnp.sum(att * sub, axis=0, keepdims=True) / sub.shape[0]


def _user_body(u_ref, im_ref, ent_ref, it_ref, r_ref, out_ref):
    it = it_ref[...]
    r = r_ref[...]
    parts = [
        _intent_vec(it[0:1], r),
        _intent_vec(it[1:2], r[0:4]),
        _intent_vec(it[2:3], r[4:8]),
        _intent_vec(it[3:4], r[8:12]),
        _intent_vec(it[4:5], r[12:16]),
    ]
    all_intent = jnp.concatenate(parts, axis=0)
    new_intent = (all_intent + it) * 0.5

    u = u_ref[...]
    score_ = jax.lax.dot_general(
        u, new_intent, (((1,), (1,)), ((), ())),
        preferred_element_type=jnp.float32)
    sm = jnp.max(score_, axis=1, keepdims=True)
    se = jnp.exp(score_ - sm)
    score = se / jnp.sum(se, axis=1, keepdims=True)

    wvec = jax.lax.dot_general(
        score, new_intent, (((1,), (0,)), ((), ())),
        preferred_element_type=jnp.float32)

    agg = jax.lax.dot_general(
        im_ref[...], ent_ref[...], (((1,), (0,)), ((), ())),
        preferred_element_type=jnp.float32)
    out_ref[...] = agg * (1.0 + wvec)


def _user_agg(user_emb, interact_mat, entity_emb, intent_emb, r_emb):
    n_users = user_emb.shape[0]
    blk = 512
    grid = n_users // blk
    return pl.pallas_call(
        _user_body,
        grid=(grid,),
        in_specs=[
            pl.BlockSpec((blk, _EMB), lambda i: (i, 0)),
            pl.BlockSpec((blk, _N_ENT), lambda i: (i, 0)),
            pl.BlockSpec((_N_ENT, _EMB), lambda i: (0, 0)),
            pl.BlockSpec((5, _EMB), lambda i: (0, 0)),
            pl.BlockSpec((16, _EMB), lambda i: (0, 0)),
        ],
        out_specs=pl.BlockSpec((blk, _EMB), lambda i: (i, 0)),
        out_shape=jax.ShapeDtypeStruct((n_users, _EMB), jnp.float32),
    )(user_emb, interact_mat, entity_emb, intent_emb, r_emb)


def kernel(entity_emb, user_emb, intent_emb, edge_index, edge_type, interact_mat, r_emb):
    npad = _EPT_PAD - _EPT
    head = edge_index[0].astype(jnp.int32).reshape(_NT, _EPT)
    tail = edge_index[1].astype(jnp.int32).reshape(_NT, _EPT)
    rel = ((edge_type.astype(jnp.int32) - 1) & 15).reshape(_NT, _EPT)
    head = jnp.concatenate(
        [head, jnp.full((_NT, npad), _N_ENT, jnp.int32)], axis=1
    ).reshape(_NT, _NTOT, _CHUNK)
    tail = jnp.concatenate(
        [tail, jnp.zeros((_NT, npad), jnp.int32)], axis=1
    ).reshape(_NT, _NTOT, _CHUNK)
    rel = jnp.concatenate(
        [rel, jnp.zeros((_NT, npad), jnp.int32)], axis=1
    ).reshape(_NT, _NTOT, _CHUNK)

    ent_halves = jnp.stack([entity_emb[:, :_HALF], entity_emb[:, _HALF:]])
    remb_halves = jnp.stack([r_emb[:, :_HALF], r_emb[:, _HALF:]])

    psum, pcnt = _edge_agg(head, tail, rel, ent_halves, remb_halves)
    entity_agg = _finalize(psum, pcnt)[:_N_ENT]
    user_agg = _user_agg(user_emb, interact_mat, entity_emb, intent_emb, r_emb)
    return entity_agg, user_agg


# speedup vs baseline: 5.9053x; 1.3838x over previous
import functools

import jax
import jax.numpy as jnp
from jax import lax
from jax.experimental import pallas as pl
from jax.experimental.pallas import tpu as pltpu
from jax.experimental.pallas import tpu_sc as plsc

_N_ENT = 10000
_N_ENT_PAD = 10240
_EMB = 128
_HALF = 64
_N_EDGE = 320000
_NT = 16
_EPT = _N_EDGE // _NT
_CHUNK = 128
_NPROC = 158
_NTOT = 160
_EPT_PAD = _NTOT * _CHUNK
_ROWS_PER_TILE = _N_ENT_PAD // _NT
_WCHUNK = 128
_NWCHUNK = _ROWS_PER_TILE // _WCHUNK


def _make_edge_kernel():
    mesh = plsc.VectorSubcoreMesh(core_axis_name="c", subcore_axis_name="s")

    @functools.partial(
        pl.kernel,
        out_type=(
            jax.ShapeDtypeStruct((2, _N_ENT_PAD, _HALF), jnp.float32),
            jax.ShapeDtypeStruct((2, _N_ENT_PAD), jnp.float32),
        ),
        mesh=mesh,
        scratch_types=[
            pltpu.VMEM((_NTOT, _CHUNK), jnp.int32),
            pltpu.VMEM((_NTOT, _CHUNK), jnp.int32),
            pltpu.VMEM((_NTOT, _CHUNK), jnp.int32),
            pltpu.VMEM((_CHUNK, _HALF), jnp.float32),
            pltpu.VMEM((_CHUNK, _HALF), jnp.float32),
            pltpu.VMEM((16, _HALF), jnp.float32),
            pltpu.VMEM((_CHUNK,), jnp.float32),
            pltpu.VMEM((_WCHUNK, _HALF), jnp.float32),
            pltpu.VMEM((_ROWS_PER_TILE,), jnp.float32),
            pltpu.VMEM_SHARED((_N_ENT_PAD, _HALF), jnp.float32),
            pltpu.VMEM_SHARED((_N_ENT_PAD,), jnp.float32),
            pltpu.SemaphoreType.DMA,
            pltpu.SemaphoreType.DMA,
        ],
        compiler_params=pltpu.CompilerParams(use_tc_tiling_on_sc=False),
    )
    def edge_kernel(head_hbm, tail_hbm, rel_hbm, ent_hbm, remb_hbm,
                    psum_hbm, pcnt_hbm,
                    hd_v, tl_v, rl_v, rows_a, rows_b, remb_v, ones_v, stage_v,
                    cnt_v, acc_sum, acc_cnt, sem_a, sem_b):
        c = lax.axis_index("c")
        s = lax.axis_index("s")
        base = s * _ROWS_PER_TILE

        pltpu.sync_copy(remb_hbm.at[c], remb_v)
        pltpu.sync_copy(head_hbm.at[s], hd_v)
        pltpu.sync_copy(tail_hbm.at[s], tl_v)
        pltpu.sync_copy(rel_hbm.at[s], rl_v)

        for i in range(_CHUNK // 16):
            ones_v[pl.ds(i * 16, 16)] = jnp.ones((16,), jnp.float32)

        def _zero_stage(i, _):
            for j in range(_HALF // 16):
                stage_v[i, pl.ds(j * 16, 16)] = jnp.zeros((16,), jnp.float32)
            return 0
        lax.fori_loop(0, _WCHUNK, _zero_stage, 0)

        def _zero_cnt(i, _):
            cnt_v[pl.ds(i * 16, 16)] = jnp.zeros((16,), jnp.float32)
            return 0
        lax.fori_loop(0, _ROWS_PER_TILE // 16, _zero_cnt, 0)

        for i in range(_NWCHUNK):
            pltpu.sync_copy(stage_v, acc_sum.at[pl.ds(base + i * _WCHUNK, _WCHUNK)])
        pltpu.sync_copy(cnt_v, acc_cnt.at[pl.ds(base, _ROWS_PER_TILE)])

        plsc.subcore_barrier()

        pltpu.async_copy(ent_hbm.at[c].at[tl_v.at[0]], rows_a, sem_a)
        pltpu.async_copy(ent_hbm.at[c].at[tl_v.at[1]], rows_b, sem_b)

        def _pair(k2, _):
            for buf, sem, off in ((rows_a, sem_a, 0), (rows_b, sem_b, 1)):
                kc = 2 * k2 + off
                pltpu.make_async_copy(ent_hbm.at[c].at[tl_v.at[kc]], buf, sem).wait()

                def _group(g, _, buf=buf, kc=kc):
                    relv = rl_v[kc, pl.ds(g * 16, 16)]
                    e0 = g * 16
                    nj = _HALF // 16
                    for l in range(16):
                        r = relv[l]
                        e = e0 + l
                        a = [buf[e, pl.ds(j * 16, 16)] for j in range(nj)]
                        b = [remb_v[r, pl.ds(j * 16, 16)] for j in range(nj)]
                        for j in range(nj):
                            buf[e, pl.ds(j * 16, 16)] = a[j] * b[j]
                    return 0
                lax.fori_loop(0, _CHUNK // 16, _group, 0)

                pltpu.sync_copy(buf, acc_sum.at[hd_v.at[kc]], add=True)
                pltpu.sync_copy(ones_v, acc_cnt.at[hd_v.at[kc]], add=True)
                pltpu.async_copy(ent_hbm.at[c].at[tl_v.at[kc + 2]], buf, sem)
            return 0
        lax.fori_loop(0, _NPROC // 2, _pair, 0)

        pltpu.make_async_copy(ent_hbm.at[c].at[tl_v.at[_NPROC]], rows_a, sem_a).wait()
        pltpu.make_async_copy(ent_hbm.at[c].at[tl_v.at[_NPROC + 1]], rows_b, sem_b).wait()

        plsc.subcore_barrier()

        for i in range(_NWCHUNK):
            off = base + i * _WCHUNK
            pltpu.sync_copy(acc_sum.at[pl.ds(off, _WCHUNK)], stage_v)
            pltpu.sync_copy(stage_v, psum_hbm.at[c, pl.ds(off, _WCHUNK)])
        pltpu.sync_copy(acc_cnt.at[pl.ds(base, _ROWS_PER_TILE)], cnt_v)
        pltpu.sync_copy(cnt_v, pcnt_hbm.at[c, pl.ds(base, _ROWS_PER_TILE)])

    return edge_kernel


_edge_agg = _make_edge_kernel()


def _finalize_body(ps_ref, pc_ref, out_ref):
    sums = jnp.concatenate([ps_ref[0], ps_ref[1]], axis=1)
    out_ref[...] = sums / jnp.maximum(pc_ref[...], 1.0)


def _finalize(psum, pcnt):
    blk = 2048
    grid = _N_ENT_PAD // blk
    return pl.pallas_call(
        _finalize_body,
        grid=(grid,),
        in_specs=[
            pl.BlockSpec((2, blk, _HALF), lambda i: (0, i, 0)),
            pl.BlockSpec((blk, 1), lambda i: (i, 0)),
        ],
        out_specs=pl.BlockSpec((blk, _EMB), lambda i: (i, 0)),
        out_shape=jax.ShapeDtypeStruct((_N_ENT_PAD, _EMB), jnp.float32),
    )(psum, pcnt[0].reshape(_N_ENT_PAD, 1))


def _intent_vec(row, sub):
    logits = jnp.sum(row * sub, axis=1, keepdims=True)
    m = jnp.max(logits, axis=0, keepdims=True)
    e = jnp.exp(logits - m)
    att = e / jnp.sum(e, axis=0, keepdims=True)
    return jnp.sum(att * sub, axis=0, keepdims=True) / sub.shape[0]


def _user_body(u_ref, im_ref, ent_ref, it_ref, r_ref, out_ref):
    it = it_ref[...]
    r = r_ref[...]
    parts = [
        _intent_vec(it[0:1], r),
        _intent_vec(it[1:2], r[0:4]),
        _intent_vec(it[2:3], r[4:8]),
        _intent_vec(it[3:4], r[8:12]),
        _intent_vec(it[4:5], r[12:16]),
    ]
    all_intent = jnp.concatenate(parts, axis=0)
    new_intent = (all_intent + it) * 0.5

    u = u_ref[...]
    score_ = jax.lax.dot_general(
        u, new_intent, (((1,), (1,)), ((), ())),
        preferred_element_type=jnp.float32)
    sm = jnp.max(score_, axis=1, keepdims=True)
    se = jnp.exp(score_ - sm)
    score = se / jnp.sum(se, axis=1, keepdims=True)

    wvec = jax.lax.dot_general(
        score, new_intent, (((1,), (0,)), ((), ())),
        preferred_element_type=jnp.float32)

    agg = jax.lax.dot_general(
        im_ref[...], ent_ref[...], (((1,), (0,)), ((), ())),
        preferred_element_type=jnp.float32)
    out_ref[...] = agg * (1.0 + wvec)


def _user_agg(user_emb, interact_mat, entity_emb, intent_emb, r_emb):
    n_users = user_emb.shape[0]
    blk = 512
    grid = n_users // blk
    return pl.pallas_call(
        _user_body,
        grid=(grid,),
        in_specs=[
            pl.BlockSpec((blk, _EMB), lambda i: (i, 0)),
            pl.BlockSpec((blk, _N_ENT), lambda i: (i, 0)),
            pl.BlockSpec((_N_ENT, _EMB), lambda i: (0, 0)),
            pl.BlockSpec((5, _EMB), lambda i: (0, 0)),
            pl.BlockSpec((16, _EMB), lambda i: (0, 0)),
        ],
        out_specs=pl.BlockSpec((blk, _EMB), lambda i: (i, 0)),
        out_shape=jax.ShapeDtypeStruct((n_users, _EMB), jnp.float32),
    )(user_emb, interact_mat, entity_emb, intent_emb, r_emb)


def kernel(entity_emb, user_emb, intent_emb, edge_index, edge_type, interact_mat, r_emb):
    npad = _EPT_PAD - _EPT
    head = edge_index[0].astype(jnp.int32).reshape(_NT, _EPT)
    tail = edge_index[1].astype(jnp.int32).reshape(_NT, _EPT)
    rel = ((edge_type.astype(jnp.int32) - 1) & 15).reshape(_NT, _EPT)
    head = jnp.concatenate(
        [head, jnp.full((_NT, npad), _N_ENT, jnp.int32)], axis=1
    ).reshape(_NT, _NTOT, _CHUNK)
    tail = jnp.concatenate(
        [tail, jnp.zeros((_NT, npad), jnp.int32)], axis=1
    ).reshape(_NT, _NTOT, _CHUNK)
    rel = jnp.concatenate(
        [rel, jnp.zeros((_NT, npad), jnp.int32)], axis=1
    ).reshape(_NT, _NTOT, _CHUNK)

    ent_halves = jnp.stack([entity_emb[:, :_HALF], entity_emb[:, _HALF:]])
    remb_halves = jnp.stack([r_emb[:, :_HALF], r_emb[:, _HALF:]])

    psum, pcnt = _edge_agg(head, tail, rel, ent_halves, remb_halves)
    entity_agg = _finalize(psum, pcnt)[:_N_ENT]
    user_agg = _user_agg(user_emb, interact_mat, entity_emb, intent_emb, r_emb)
    return entity_agg, user_agg


# speedup vs baseline: 6.0206x; 1.0195x over previous
import functools

import jax
import jax.numpy as jnp
from jax import lax
from jax.experimental import pallas as pl
from jax.experimental.pallas import tpu as pltpu
from jax.experimental.pallas import tpu_sc as plsc

_N_ENT = 10000
_N_ENT_PAD = 10240
_EMB = 128
_HALF = 64
_N_EDGE = 320000
_NT = 16
_EPT = _N_EDGE // _NT
_CHUNK = 128
_NPROC = 159
_NTOT = 161
_EPT_PAD = _NTOT * _CHUNK
_ROWS_PER_TILE = _N_ENT_PAD // _NT
_WCHUNK = 128
_NWCHUNK = _ROWS_PER_TILE // _WCHUNK


def _make_edge_kernel():
    mesh = plsc.VectorSubcoreMesh(core_axis_name="c", subcore_axis_name="s")

    @functools.partial(
        pl.kernel,
        out_type=(
            jax.ShapeDtypeStruct((2, _N_ENT_PAD, _HALF), jnp.float32),
            jax.ShapeDtypeStruct((2, _N_ENT_PAD), jnp.float32),
        ),
        mesh=mesh,
        scratch_types=[
            pltpu.VMEM((_NTOT, _CHUNK), jnp.int32),
            pltpu.VMEM((3, _CHUNK), jnp.int32),
            pltpu.VMEM((3, _CHUNK), jnp.int32),
            pltpu.VMEM((3, _CHUNK), jnp.int32),
            pltpu.VMEM((_CHUNK, _HALF), jnp.float32),
            pltpu.VMEM((_CHUNK, _HALF), jnp.float32),
            pltpu.VMEM((_CHUNK, _HALF), jnp.float32),
            pltpu.VMEM((16, _HALF), jnp.float32),
            pltpu.VMEM((_CHUNK,), jnp.float32),
            pltpu.VMEM((_WCHUNK, _HALF), jnp.float32),
            pltpu.VMEM((_ROWS_PER_TILE,), jnp.float32),
            pltpu.VMEM_SHARED((_N_ENT_PAD, _HALF), jnp.float32),
            pltpu.VMEM_SHARED((_N_ENT_PAD,), jnp.float32),
            pltpu.SemaphoreType.DMA,
            pltpu.SemaphoreType.DMA,
            pltpu.SemaphoreType.DMA,
            pltpu.SemaphoreType.DMA,
            pltpu.SemaphoreType.DMA,
            pltpu.SemaphoreType.DMA,
        ],
        compiler_params=pltpu.CompilerParams(use_tc_tiling_on_sc=False),
    )
    def edge_kernel(packed_hbm, ent_hbm, remb_hbm,
                    psum_hbm, pcnt_hbm,
                    pk_v, tl_s, hd_s, rl_s, rows_a, rows_b, rows_c, remb_v,
                    ones_v, stage_v, cnt_v, acc_sum, acc_cnt,
                    gsem_a, gsem_b, gsem_c, ssem_a, ssem_b, ssem_c):
        c = lax.axis_index("c")
        s = lax.axis_index("s")
        base = s * _ROWS_PER_TILE

        pltpu.sync_copy(remb_hbm.at[c], remb_v)
        pltpu.sync_copy(packed_hbm.at[s], pk_v)

        for i in range(_CHUNK // 16):
            ones_v[pl.ds(i * 16, 16)] = jnp.ones((16,), jnp.float32)

        def _zero_stage(i, _):
            for j in range(_HALF // 16):
                stage_v[i, pl.ds(j * 16, 16)] = jnp.zeros((16,), jnp.float32)
            return 0
        lax.fori_loop(0, _WCHUNK, _zero_stage, 0)

        def _zero_cnt(i, _):
            cnt_v[pl.ds(i * 16, 16)] = jnp.zeros((16,), jnp.float32)
            return 0
        lax.fori_loop(0, _ROWS_PER_TILE // 16, _zero_cnt, 0)

        for i in range(_NWCHUNK):
            pltpu.sync_copy(stage_v, acc_sum.at[pl.ds(base + i * _WCHUNK, _WCHUNK)])
        pltpu.sync_copy(cnt_v, acc_cnt.at[pl.ds(base, _ROWS_PER_TILE)])

        plsc.subcore_barrier()

        rows = (rows_a, rows_b, rows_c)
        gsems = (gsem_a, gsem_b, gsem_c)
        ssems = (ssem_a, ssem_b, ssem_c)

        def _unpack(kc, j):
            for g in range(_CHUNK // 16):
                sl = pl.ds(g * 16, 16)
                p = pk_v[kc, sl]
                tl_s[j, sl] = p & 0x3FFF
                hd_s[j, sl] = (p >> 14) & 0x3FFF
                rl_s[j, sl] = (p >> 28) & 0xF

        def _gather(j):
            pltpu.async_copy(ent_hbm.at[c].at[tl_s.at[j]], rows[j], gsems[j])

        def _wait_gather(j):
            pltpu.make_async_copy(
                ent_hbm.at[c].at[tl_s.at[j]], rows[j], gsems[j]).wait()

        def _mult(j):
            buf = rows[j]

            def _group(g, _):
                relv = rl_s[j, pl.ds(g * 16, 16)]
                e0 = g * 16
                nj = _HALF // 16
                for l in range(16):
                    r = relv[l]
                    e = e0 + l
                    a = [buf[e, pl.ds(jj * 16, 16)] for jj in range(nj)]
                    b = [remb_v[r, pl.ds(jj * 16, 16)] for jj in range(nj)]
                    for jj in range(nj):
                        buf[e, pl.ds(jj * 16, 16)] = a[jj] * b[jj]
                return 0
            lax.fori_loop(0, _CHUNK // 16, _group, 0)

        def _scatter(j):
            pltpu.async_copy(rows[j], acc_sum.at[hd_s.at[j]], ssems[j], add=True)
            pltpu.async_copy(ones_v, acc_cnt.at[hd_s.at[j]], ssems[j], add=True)

        def _wait_scatter(j):
            pltpu.make_async_copy(rows[j], acc_sum.at[hd_s.at[j]], ssems[j]).wait()
            pltpu.make_async_copy(ones_v, acc_cnt.at[hd_s.at[j]], ssems[j]).wait()

        for j in range(3):
            _unpack(j, j)
        for j in range(3):
            _gather(j)
        for j in range(3):
            _wait_gather(j)
            _mult(j)
            _scatter(j)
            if j > 0:
                _wait_scatter(j - 1)
                _unpack(j + 2, j - 1)
                _gather(j - 1)

        def _tri(k, _):
            base_kc = 3 * k + 3
            for j in range(3):
                kc = base_kc + j
                pj = (j - 1) % 3
                _wait_gather(j)
                _mult(j)
                _scatter(j)
                _wait_scatter(pj)
                _unpack(kc + 2, pj)
                _gather(pj)
            return 0
        lax.fori_loop(0, (_NPROC - 3) // 3, _tri, 0)

        _wait_scatter(2)
        _wait_gather(0)
        _wait_gather(1)

        plsc.subcore_barrier()

        for i in range(_NWCHUNK):
            off = base + i * _WCHUNK
            pltpu.sync_copy(acc_sum.at[pl.ds(off, _WCHUNK)], stage_v)
            pltpu.sync_copy(stage_v, psum_hbm.at[c, pl.ds(off, _WCHUNK)])
        pltpu.sync_copy(acc_cnt.at[pl.ds(base, _ROWS_PER_TILE)], cnt_v)
        pltpu.sync_copy(cnt_v, pcnt_hbm.at[c, pl.ds(base, _ROWS_PER_TILE)])

    return edge_kernel


_edge_agg = _make_edge_kernel()


def _finalize_body(ps_ref, pc_ref, out_ref):
    sums = jnp.concatenate([ps_ref[0], ps_ref[1]], axis=1)
    out_ref[...] = sums / jnp.maximum(pc_ref[...], 1.0)


def _finalize(psum, pcnt):
    blk = 2048
    grid = _N_ENT_PAD // blk
    return pl.pallas_call(
        _finalize_body,
        grid=(grid,),
        in_specs=[
            pl.BlockSpec((2, blk, _HALF), lambda i: (0, i, 0)),
            pl.BlockSpec((blk, 1), lambda i: (i, 0)),
        ],
        out_specs=pl.BlockSpec((blk, _EMB), lambda i: (i, 0)),
        out_shape=jax.ShapeDtypeStruct((_N_ENT_PAD, _EMB), jnp.float32),
    )(psum, pcnt[0].reshape(_N_ENT_PAD, 1))


def _intent_vec(row, sub):
    logits = jnp.sum(row * sub, axis=1, keepdims=True)
    m = jnp.max(logits, axis=0, keepdims=True)
    e = jnp.exp(logits - m)
    att = e / jnp.sum(e, axis=0, keepdims=True)
    return jnp.sum(att * sub, axis=0, keepdims=True) / sub.shape[0]


def _user_body(u_ref, im_ref, ent_ref, it_ref, r_ref, out_ref):
    it = it_ref[...]
    r = r_ref[...]
    parts = [
        _intent_vec(it[0:1], r),
        _intent_vec(it[1:2], r[0:4]),
        _intent_vec(it[2:3], r[4:8]),
        _intent_vec(it[3:4], r[8:12]),
        _intent_vec(it[4:5], r[12:16]),
    ]
    all_intent = jnp.concatenate(parts, axis=0)
    new_intent = (all_intent + it) * 0.5

    u = u_ref[...]
    score_ = jax.lax.dot_general(
        u, new_intent, (((1,), (1,)), ((), ())),
        preferred_element_type=jnp.float32)
    sm = jnp.max(score_, axis=1, keepdims=True)
    se = jnp.exp(score_ - sm)
    score = se / jnp.sum(se, axis=1, keepdims=True)

    wvec = jax.lax.dot_general(
        score, new_intent, (((1,), (0,)), ((), ())),
        preferred_element_type=jnp.float32)

    agg = jax.lax.dot_general(
        im_ref[...], ent_ref[...], (((1,), (0,)), ((), ())),
        preferred_element_type=jnp.float32)
    out_ref[...] = agg * (1.0 + wvec)


def _user_agg(user_emb, interact_mat, entity_emb, intent_emb, r_emb):
    n_users = user_emb.shape[0]
    blk = 512
    grid = n_users // blk
    return pl.pallas_call(
        _user_body,
        grid=(grid,),
        in_specs=[
            pl.BlockSpec((blk, _EMB), lambda i: (i, 0)),
            pl.BlockSpec((blk, _N_ENT), lambda i: (i, 0)),
            pl.BlockSpec((_N_ENT, _EMB), lambda i: (0, 0)),
            pl.BlockSpec((5, _EMB), lambda i: (0, 0)),
            pl.BlockSpec((16, _EMB), lambda i: (0, 0)),
        ],
        out_specs=pl.BlockSpec((blk, _EMB), lambda i: (i, 0)),
        out_shape=jax.ShapeDtypeStruct((n_users, _EMB), jnp.float32),
    )(user_emb, interact_mat, entity_emb, intent_emb, r_emb)


def kernel(entity_emb, user_emb, intent_emb, edge_index, edge_type, interact_mat, r_emb):
    npad = _EPT_PAD - _EPT
    head = edge_index[0].astype(jnp.int32)
    tail = edge_index[1].astype(jnp.int32)
    rel = (edge_type.astype(jnp.int32) - 1) & 15
    packed = (tail | (head << 14) | (rel << 28)).reshape(_NT, _EPT)
    packed = jnp.concatenate(
        [packed, jnp.full((_NT, npad), _N_ENT << 14, jnp.int32)], axis=1
    ).reshape(_NT, _NTOT, _CHUNK)

    ent_halves = jnp.stack([entity_emb[:, :_HALF], entity_emb[:, _HALF:]])
    remb_halves = jnp.stack([r_emb[:, :_HALF], r_emb[:, _HALF:]])

    psum, pcnt = _edge_agg(packed, ent_halves, remb_halves)
    entity_agg = _finalize(psum, pcnt)[:_N_ENT]
    user_agg = _user_agg(user_emb, interact_mat, entity_emb, intent_emb, r_emb)
    return entity_agg, user_agg
